# baseline (speedup 1.0000x reference)
import functools

import jax
import jax.numpy as jnp
from jax import lax
from jax.experimental import pallas as pl
from jax.experimental.pallas import tpu as pltpu
from jax.experimental.pallas import tpu_sc as plsc

_TABLE_SIZE = 1000000
_K = 8
_BATCH = 16384

_NC = 2
_NS = 16
_NW = _NC * _NS
_BPW = _BATCH // _NW
_SUB = 16
_NSUB = _BPW // _SUB
_TW = 128


@functools.partial(
    pl.kernel,
    out_type=(
        jax.ShapeDtypeStruct((_K, _BATCH), jnp.int32),
        jax.ShapeDtypeStruct((_K, _BATCH), jnp.float32),
    ),
    mesh=plsc.VectorSubcoreMesh(core_axis_name="c", subcore_axis_name="s"),
    scratch_types=[
        pltpu.VMEM((_BPW,), jnp.int32),
        pltpu.VMEM((_K, _SUB * _TW), jnp.int32),
        pltpu.VMEM((_K, _SUB * _TW), jnp.int32),
        pltpu.VMEM((_K, _SUB * _TW), jnp.float32),
        pltpu.VMEM((_K, _SUB * _TW), jnp.float32),
        pltpu.VMEM((_K, _TW), jnp.int32),
        pltpu.VMEM((_K, _TW), jnp.float32),
        pltpu.SemaphoreType.DMA,
        pltpu.SemaphoreType.DMA,
        pltpu.SemaphoreType.DMA,
        pltpu.SemaphoreType.DMA,
    ],
    compiler_params=pltpu.CompilerParams(
        disable_bounds_checks=True, needs_layout_passes=False),
)
def _gather_kernel(idx_hbm, cand_hbm, prob_hbm, cand_out, prob_out,
                   idx_v, blk_c0, blk_c1, blk_p0, blk_p1, stg_c, stg_p,
                   sem_c0, sem_c1, sem_p0, sem_p1):
    wid = lax.axis_index("s") * _NC + lax.axis_index("c")
    base = wid * _BPW
    pltpu.sync_copy(idx_hbm.at[pl.ds(base, _BPW)], idx_v)
    lanes = lax.iota(jnp.int32, 16)
    blks = ((blk_c0, blk_p0, sem_c0, sem_p0), (blk_c1, blk_p1, sem_c1, sem_p1))

    def fire(s, bc, bp, sc_, sp_):
        v = idx_v[pl.ds(s * _SUB, 16)]
        t = lax.shift_right_logical(v, 7)
        for j in range(16):
            col = t[j] * _TW
            d = j * _TW
            pltpu.async_copy(cand_hbm.at[:, pl.ds(col, _TW)],
                             bc.at[:, pl.ds(d, _TW)], sc_)
            pltpu.async_copy(prob_hbm.at[:, pl.ds(col, _TW)],
                             bp.at[:, pl.ds(d, _TW)], sp_)

    def drain(bc, bp, sc_, sp_):
        pltpu.make_async_copy(cand_hbm.at[:, pl.ds(0, _SUB * _TW)], bc,
                              sc_).wait()
        pltpu.make_async_copy(prob_hbm.at[:, pl.ds(0, _SUB * _TW)], bp,
                              sp_).wait()

    def extract(s, bc, bp):
        v = idx_v[pl.ds(s * _SUB, 16)]
        col = lanes * _TW + (v & jnp.int32(_TW - 1))
        off = (s % 8) * _SUB
        for k in range(_K):
            row = jnp.full((16,), k, jnp.int32)
            stg_c[k, pl.ds(off, 16)] = plsc.load_gather(bc, [row, col])
            stg_p[k, pl.ds(off, 16)] = plsc.load_gather(bp, [row, col])

    def flush(s):
        @pl.when(s % 8 == 7)
        def _():
            out_base = base + (s // 8) * _TW
            pltpu.sync_copy(stg_c, cand_out.at[:, pl.ds(out_base, _TW)])
            pltpu.sync_copy(stg_p, prob_out.at[:, pl.ds(out_base, _TW)])

    fire(0, *blks[0])

    def pair_body(i, _):
        s0 = 2 * i
        fire(s0 + 1, *blks[1])
        drain(*blks[0])
        extract(s0, blks[0][0], blks[0][1])
        flush(s0)

        @pl.when(s0 + 2 < _NSUB)
        def _():
            fire(s0 + 2, *blks[0])
        drain(*blks[1])
        extract(s0 + 1, blks[1][0], blks[1][1])
        flush(s0 + 1)
        return 0

    lax.fori_loop(0, _NSUB // 2, pair_body, 0)


def kernel(indices, candidates, probs):
    cand_t, prob_t = _gather_kernel(indices, candidates.T, probs.T)
    return cand_t.T, prob_t.T
